# baseline (speedup 1.0000x reference)
import jax
import jax.numpy as jnp
from jax import lax
from jax.experimental import pallas as pl
from jax.experimental.pallas import tpu as pltpu
from jax.experimental.pallas import tpu_sc as plsc

_F = 26
_HF = 13
_V = 100000
_D = 32
_B = 16384

_L = 16
_NW = 32
_CB = _B // _NW
_GS = _L
_GROUPS = _CB // _GS
_CHUNK = 104
_CPG = _GS * _HF // _CHUNK
_RG = _GS * _HF
_CPW = _CB * _HF // _CHUNK
_NP = _D + 2


def _make_half_kernel(second):
    scmesh = plsc.VectorSubcoreMesh(core_axis_name="c", subcore_axis_name="s")

    def body(*refs):
        if second:
            (dma_hbm, raw_hbm, emb_hbm, lin_hbm, part_hbm, bias_hbm, out_hbm,
             dma_v, raw_v, rows_v, lin_v, part_v, out_v, bias_v,
             sem_e0, sem_e1, sem_l0, sem_l1) = refs
        else:
            (dma_hbm, raw_hbm, emb_hbm, lin_hbm, out_hbm,
             dma_v, raw_v, rows_v, lin_v, out_v,
             sem_e0, sem_e1, sem_l0, sem_l1) = refs
        c = lax.axis_index("c")
        s = lax.axis_index("s")
        wid = s * 2 + c
        pltpu.sync_copy(dma_hbm.at[pl.ds(wid * _CPW, _CPW), :], dma_v)
        if second:
            pltpu.sync_copy(part_hbm.at[:, pl.ds(wid * _CB, _CB)], part_v)
            pltpu.sync_copy(bias_hbm, bias_v)
        sems_e = (sem_e0, sem_e1)
        sems_l = (sem_l0, sem_l1)

        def fire(g, par):
            pltpu.sync_copy(raw_hbm.at[pl.ds(wid * _CPW + g * _CPG, _CPG), :],
                            raw_v.at[par])
            for j in range(_CPG):
                ch = g * _CPG + j
                pltpu.async_copy(emb_hbm.at[dma_v.at[ch]],
                                 rows_v.at[par].at[pl.ds(j * _CHUNK, _CHUNK), :],
                                 sems_e[par])
                pltpu.async_copy(lin_hbm.at[raw_v.at[par].at[j]],
                                 lin_v.at[par].at[j],
                                 sems_l[par])

        def drain(par):
            for j in range(_CPG):
                pltpu.make_async_copy(emb_hbm.at[pl.ds(0, _CHUNK), :],
                                      rows_v.at[par].at[pl.ds(j * _CHUNK, _CHUNK), :],
                                      sems_e[par]).wait()
                pltpu.make_async_copy(lin_hbm.at[pl.ds(0, _CHUNK)],
                                      lin_v.at[par].at[j],
                                      sems_l[par]).wait()

        iota13 = lax.iota(jnp.int32, _L) * _HF
        zero = jnp.zeros((_L,), jnp.float32)

        def compute(g, par):
            rows2 = rows_v.at[par]
            raw2 = raw_v.at[par]
            lin2 = lin_v.at[par]

            def fbody(f, carry):
                accs = carry[:_D]
                acc_sq = carry[_D]
                lin_acc = carry[_D + 1]
                r = iota13 + f
                rc = r // _CHUNK
                rw = r % _CHUNK
                ivraw = plsc.load_gather(raw2, [rc, rw])
                colb = (ivraw & 3) * _D
                lin_acc = lin_acc + plsc.load_gather(lin2, [rc, rw])
                new_accs = []
                for d in range(_D):
                    v = plsc.load_gather(rows2, [r, colb + d])
                    new_accs.append(accs[d] + v)
                    acc_sq = acc_sq + v * v
                return (*new_accs, acc_sq, lin_acc)

            if second:
                init = tuple(part_v[d, pl.ds(g * _GS, _L)] for d in range(_NP))
            else:
                init = (zero,) * _NP
            res = lax.fori_loop(0, _HF, fbody, init)
            accs, acc_sq, lin_acc = res[:_D], res[_D], res[_D + 1]
            if second:
                ss = accs[0] * accs[0]
                for d in range(1, _D):
                    ss = ss + accs[d] * accs[d]
                logit = bias_v[...] + lin_acc + 0.5 * (ss - acc_sq)
                out_v[pl.ds(g * _GS, _L)] = 1.0 / (1.0 + jnp.exp(-logit))
            else:
                for d in range(_D):
                    out_v[d, pl.ds(g * _GS, _L)] = accs[d]
                out_v[_D, pl.ds(g * _GS, _L)] = acc_sq
                out_v[_D + 1, pl.ds(g * _GS, _L)] = lin_acc

        fire(0, 0)

        def gbody(g2, carry):
            fire(2 * g2 + 1, 1)
            drain(0)
            compute(2 * g2, 0)

            @pl.when(g2 < _GROUPS // 2 - 1)
            def _():
                fire(2 * g2 + 2, 0)

            drain(1)
            compute(2 * g2 + 1, 1)
            return carry

        lax.fori_loop(0, _GROUPS // 2, gbody, 0)
        if second:
            pltpu.sync_copy(out_v, out_hbm.at[pl.ds(wid * _CB, _CB)])
        else:
            pltpu.sync_copy(out_v, out_hbm.at[:, pl.ds(wid * _CB, _CB)])

    scratch = [
        pltpu.VMEM((_CPW, _CHUNK), jnp.int32),
        pltpu.VMEM((2, _CPG, _CHUNK), jnp.int32),
        pltpu.VMEM((2, _RG, 4 * _D), jnp.float32),
        pltpu.VMEM((2, _CPG, _CHUNK), jnp.float32),
    ]
    if second:
        scratch.append(pltpu.VMEM((_NP, _CB), jnp.float32))
        scratch.append(pltpu.VMEM((_CB,), jnp.float32))
        scratch.append(pltpu.VMEM((_L,), jnp.float32))
        out_type = jax.ShapeDtypeStruct((_B,), jnp.float32)
    else:
        scratch.append(pltpu.VMEM((_NP, _CB), jnp.float32))
        out_type = jax.ShapeDtypeStruct((_NP, _B), jnp.float32)
    scratch += [pltpu.SemaphoreType.DMA] * 4

    return pl.kernel(
        body,
        out_type=out_type,
        mesh=scmesh,
        scratch_types=scratch,
        compiler_params=pltpu.CompilerParams(needs_layout_passes=False),
    )


def kernel(x, emb_tables, lin_tables, bias):
    F, V, D = emb_tables.shape
    offs = jnp.arange(_HF, dtype=jnp.int32)[None, :] * V

    idx_a = (x[:, :_HF] + offs).reshape(-1)
    idx_b = (x[:, _HF:] + offs).reshape(-1)
    emb_a = emb_tables[:_HF].reshape(_HF * V // 4, 4 * D)
    emb_b = emb_tables[_HF:].reshape(_HF * V // 4, 4 * D)
    lin_a = lin_tables[:_HF].reshape(_HF * V)
    lin_b = lin_tables[_HF:].reshape(_HF * V)
    bias16 = jnp.broadcast_to(bias.astype(jnp.float32), (_L,))

    part = _make_half_kernel(False)(
        (idx_a >> 2).reshape(-1, _CHUNK), idx_a.reshape(-1, _CHUNK),
        emb_a, lin_a)
    out = _make_half_kernel(True)(
        (idx_b >> 2).reshape(-1, _CHUNK), idx_b.reshape(-1, _CHUNK),
        emb_b, lin_b, part, bias16)
    return out.reshape(-1, 1)


# speedup vs baseline: 1.3480x; 1.3480x over previous
import jax
import jax.numpy as jnp
from jax import lax
from jax.experimental import pallas as pl
from jax.experimental.pallas import tpu as pltpu
from jax.experimental.pallas import tpu_sc as plsc

_F = 26
_V = 100000
_D = 32
_B = 16384

_L = 16
_NW = 32
_CB = _B // _NW
_GS = _L
_GROUPS = _CB // _GS
_EPG = _GS * _F * _D
_ECH = 128
_ECPG = _EPG // _ECH
_LCH = 104
_LCPG = _GS * _F // _LCH
_LCPW = _CB * _F // _LCH


def _make_fm_kernel():
    scmesh = plsc.VectorSubcoreMesh(core_axis_name="c", subcore_axis_name="s")

    def body(eidx_hbm, raw_hbm, emb_hbm, lin_hbm, bias_hbm, out_hbm,
             eidx_v, raw_v, rows_v, lin_v, out_v, bias_v,
             sem_i0, sem_i1, sem_e0, sem_e1, sem_l0, sem_l1):
        c = lax.axis_index("c")
        s = lax.axis_index("s")
        wid = s * 2 + c
        pltpu.sync_copy(bias_hbm, bias_v)
        sems_i = (sem_i0, sem_i1)
        sems_e = (sem_e0, sem_e1)
        sems_l = (sem_l0, sem_l1)
        ebase = wid * _GROUPS * _ECPG

        def stage(g, par):
            pltpu.async_copy(
                eidx_hbm.at[pl.ds(ebase + g * _ECPG, _ECPG), :],
                eidx_v.at[par], sems_i[par])
            pltpu.sync_copy(
                raw_hbm.at[pl.ds(wid * _LCPW + g * _LCPG, _LCPG), :],
                raw_v.at[par])

        def fire(g, par):
            pltpu.make_async_copy(eidx_hbm.at[pl.ds(0, _ECPG), :],
                                  eidx_v.at[par], sems_i[par]).wait()

            def echunk(j, carry):
                pltpu.async_copy(emb_hbm.at[eidx_v.at[par].at[j]],
                                 rows_v.at[par].at[j], sems_e[par])
                return carry

            lax.fori_loop(0, _ECPG, echunk, 0)
            for j in range(_LCPG):
                pltpu.async_copy(lin_hbm.at[raw_v.at[par].at[j]],
                                 lin_v.at[par].at[j], sems_l[par])

        def drain(par):
            def ewait(j, carry):
                pltpu.make_async_copy(emb_hbm.at[pl.ds(0, _ECH)],
                                      rows_v.at[par].at[0], sems_e[par]).wait()
                return carry

            lax.fori_loop(0, _ECPG, ewait, 0)
            for j in range(_LCPG):
                pltpu.make_async_copy(lin_hbm.at[pl.ds(0, _LCH)],
                                      lin_v.at[par].at[j],
                                      sems_l[par]).wait()

        iota = lax.iota(jnp.int32, _L)
        iota832 = iota * (_F * _D)
        iota26 = iota * _F
        zero = jnp.zeros((_L,), jnp.float32)

        def compute(g, par):
            rows2 = rows_v.at[par]
            raw2 = raw_v.at[par]
            lin2 = lin_v.at[par]

            def fbody(f, carry):
                accs = carry[:_D]
                acc_sq = carry[_D]
                lin_acc = carry[_D + 1]
                r = iota26 + f
                lin_acc = lin_acc + plsc.load_gather(
                    lin2, [r // _LCH, r % _LCH])
                fbase = iota832 + f * _D
                new_accs = []
                for d in range(_D):
                    e = fbase + d
                    v = plsc.load_gather(rows2, [e >> 7, e & 127])
                    new_accs.append(accs[d] + v)
                    acc_sq = acc_sq + v * v
                return (*new_accs, acc_sq, lin_acc)

            init = (zero,) * (_D + 2)
            res = lax.fori_loop(0, _F, fbody, init)
            accs, acc_sq, lin_acc = res[:_D], res[_D], res[_D + 1]
            ss = accs[0] * accs[0]
            for d in range(1, _D):
                ss = ss + accs[d] * accs[d]
            logit = bias_v[...] + lin_acc + 0.5 * (ss - acc_sq)
            out_v[pl.ds(g * _GS, _L)] = 1.0 / (1.0 + jnp.exp(-logit))

        stage(0, 0)
        fire(0, 0)
        stage(1, 1)

        def gbody(g2, carry):
            fire(2 * g2 + 1, 1)
            drain(0)
            compute(2 * g2, 0)

            @pl.when(g2 < _GROUPS // 2 - 1)
            def _():
                stage(2 * g2 + 2, 0)
                fire(2 * g2 + 2, 0)

            drain(1)

            @pl.when(g2 < _GROUPS // 2 - 1)
            def _():
                stage(2 * g2 + 3, 1)

            compute(2 * g2 + 1, 1)
            return carry

        lax.fori_loop(0, _GROUPS // 2, gbody, 0)
        pltpu.sync_copy(out_v, out_hbm.at[pl.ds(wid * _CB, _CB)])

    return pl.kernel(
        body,
        out_type=jax.ShapeDtypeStruct((_B,), jnp.float32),
        mesh=scmesh,
        scratch_types=[
            pltpu.VMEM((2, _ECPG, _ECH), jnp.int32),
            pltpu.VMEM((2, _LCPG, _LCH), jnp.int32),
            pltpu.VMEM((2, _ECPG, _ECH), jnp.float32),
            pltpu.VMEM((2, _LCPG, _LCH), jnp.float32),
            pltpu.VMEM((_CB,), jnp.float32),
            pltpu.VMEM((_L,), jnp.float32),
            pltpu.SemaphoreType.DMA,
            pltpu.SemaphoreType.DMA,
            pltpu.SemaphoreType.DMA,
            pltpu.SemaphoreType.DMA,
            pltpu.SemaphoreType.DMA,
            pltpu.SemaphoreType.DMA,
        ],
        compiler_params=pltpu.CompilerParams(needs_layout_passes=False),
    )


def kernel(x, emb_tables, lin_tables, bias):
    F, V, D = emb_tables.shape
    p0 = x + jnp.arange(F, dtype=jnp.int32)[None, :] * (D * V)
    eidx = (p0[:, :, None]
            + jnp.arange(D, dtype=jnp.int32)[None, None, :] * V)
    eidx2d = eidx.reshape(-1, _ECH)
    raw2d = (x + jnp.arange(F, dtype=jnp.int32)[None, :] * V).reshape(-1, _LCH)
    emb_1d = emb_tables.transpose(0, 2, 1).reshape(-1)
    lin_flat = lin_tables.reshape(F * V)
    bias16 = jnp.broadcast_to(bias.astype(jnp.float32), (_L,))
    out = _make_fm_kernel()(eidx2d, raw2d, emb_1d, lin_flat, bias16)
    return out.reshape(-1, 1)


# speedup vs baseline: 1.4315x; 1.0620x over previous
import jax
import jax.numpy as jnp
from jax import lax
from jax.experimental import pallas as pl
from jax.experimental.pallas import tpu as pltpu
from jax.experimental.pallas import tpu_sc as plsc

_F = 26
_V = 100000
_D = 32
_B = 16384

_L = 16
_NW = 32
_CB = _B // _NW
_GS = _L
_GROUPS = _CB // _GS
_EPG = _GS * _F * _D
_ECH = 128
_ECPG = _EPG // _ECH
_LCH = 104
_LCPG = _GS * _F // _LCH
_LCPW = _CB * _F // _LCH


def _make_fm_kernel():
    scmesh = plsc.VectorSubcoreMesh(core_axis_name="c", subcore_axis_name="s")

    def body(p0_hbm, raw_hbm, emb_hbm, lin_hbm, bias_hbm, out_hbm,
             eidx0_v, eidx1_v, p0_v, raw_v, rows_v, lin_v, out_v, bias_v,
             sem_e0, sem_e1, sem_l0, sem_l1):
        c = lax.axis_index("c")
        s = lax.axis_index("s")
        wid = s * 2 + c
        pltpu.sync_copy(bias_hbm, bias_v)
        sems_e = (sem_e0, sem_e1)
        sems_l = (sem_l0, sem_l1)

        def stage(g, par):
            pltpu.sync_copy(p0_hbm.at[pl.ds(wid * _GROUPS + g, 1), :],
                            p0_v.at[par])
            pltpu.sync_copy(
                raw_hbm.at[pl.ds(wid * _LCPW + g * _LCPG, _LCPG), :],
                raw_v.at[par])

        iota32 = lax.iota(jnp.int32, _L) * _D
        eflat = (eidx0_v, eidx1_v)

        def build(par):
            def bb(k, carry):
                prs = p0_v[par, 0, pl.ds(k * _L, _L)]
                base = k * (_L * _D)
                for d in range(_D):
                    e_vec = iota32 + (base + d)
                    plsc.store_scatter(eflat[par], [e_vec], prs + d * _V)
                return carry

            lax.fori_loop(0, _F, bb, 0)

        def fire(g, par):
            def echunk(j, carry):
                pltpu.async_copy(
                    emb_hbm.at[eflat[par].at[pl.ds(j * _ECH, _ECH)]],
                    rows_v.at[par].at[j], sems_e[par])
                return carry

            lax.fori_loop(0, _ECPG, echunk, 0)
            for j in range(_LCPG):
                pltpu.async_copy(lin_hbm.at[raw_v.at[par].at[j]],
                                 lin_v.at[par].at[j], sems_l[par])

        def drain(par):
            def ewait(j, carry):
                pltpu.make_async_copy(emb_hbm.at[pl.ds(0, _ECH)],
                                      rows_v.at[par].at[0], sems_e[par]).wait()
                return carry

            lax.fori_loop(0, _ECPG, ewait, 0)
            for j in range(_LCPG):
                pltpu.make_async_copy(lin_hbm.at[pl.ds(0, _LCH)],
                                      lin_v.at[par].at[j],
                                      sems_l[par]).wait()

        iota = lax.iota(jnp.int32, _L)
        iota832 = iota * (_F * _D)
        iota26 = iota * _F
        zero = jnp.zeros((_L,), jnp.float32)

        def compute(g, par):
            rows2 = rows_v.at[par]
            raw2 = raw_v.at[par]
            lin2 = lin_v.at[par]

            def fbody(f, carry):
                accs = carry[:_D]
                acc_sq = carry[_D]
                lin_acc = carry[_D + 1]
                r = iota26 + f
                lin_acc = lin_acc + plsc.load_gather(
                    lin2, [r // _LCH, r % _LCH])
                fbase = iota832 + f * _D
                new_accs = []
                for d in range(_D):
                    e = fbase + d
                    v = plsc.load_gather(rows2, [e >> 7, e & 127])
                    new_accs.append(accs[d] + v)
                    acc_sq = acc_sq + v * v
                return (*new_accs, acc_sq, lin_acc)

            init = (zero,) * (_D + 2)
            res = lax.fori_loop(0, _F, fbody, init)
            accs, acc_sq, lin_acc = res[:_D], res[_D], res[_D + 1]
            ss = accs[0] * accs[0]
            for d in range(1, _D):
                ss = ss + accs[d] * accs[d]
            logit = bias_v[...] + lin_acc + 0.5 * (ss - acc_sq)
            out_v[pl.ds(g * _GS, _L)] = 1.0 / (1.0 + jnp.exp(-logit))

        stage(0, 0)
        build(0)
        fire(0, 0)
        stage(1, 1)
        build(1)

        def gbody(g2, carry):
            fire(2 * g2 + 1, 1)
            drain(0)
            compute(2 * g2, 0)

            @pl.when(g2 < _GROUPS // 2 - 1)
            def _():
                stage(2 * g2 + 2, 0)
                build(0)
                fire(2 * g2 + 2, 0)

            drain(1)

            @pl.when(g2 < _GROUPS // 2 - 1)
            def _():
                stage(2 * g2 + 3, 1)
                build(1)

            compute(2 * g2 + 1, 1)
            return carry

        lax.fori_loop(0, _GROUPS // 2, gbody, 0)
        pltpu.sync_copy(out_v, out_hbm.at[pl.ds(wid * _CB, _CB)])

    return pl.kernel(
        body,
        out_type=jax.ShapeDtypeStruct((_B,), jnp.float32),
        mesh=scmesh,
        scratch_types=[
            pltpu.VMEM((_EPG,), jnp.int32),
            pltpu.VMEM((_EPG,), jnp.int32),
            pltpu.VMEM((2, 1, _GS * _F), jnp.int32),
            pltpu.VMEM((2, _LCPG, _LCH), jnp.int32),
            pltpu.VMEM((2, _ECPG, _ECH), jnp.float32),
            pltpu.VMEM((2, _LCPG, _LCH), jnp.float32),
            pltpu.VMEM((_CB,), jnp.float32),
            pltpu.VMEM((_L,), jnp.float32),
            pltpu.SemaphoreType.DMA,
            pltpu.SemaphoreType.DMA,
            pltpu.SemaphoreType.DMA,
            pltpu.SemaphoreType.DMA,
        ],
        compiler_params=pltpu.CompilerParams(needs_layout_passes=False),
    )


def kernel(x, emb_tables, lin_tables, bias):
    F, V, D = emb_tables.shape
    p0 = (x + jnp.arange(F, dtype=jnp.int32)[None, :] * (D * V))
    p0_2d = p0.reshape(-1, _GS * _F)
    raw2d = (x + jnp.arange(F, dtype=jnp.int32)[None, :] * V).reshape(-1, _LCH)
    emb_1d = emb_tables.transpose(0, 2, 1).reshape(-1)
    lin_flat = lin_tables.reshape(F * V)
    bias16 = jnp.broadcast_to(bias.astype(jnp.float32), (_L,))
    out = _make_fm_kernel()(p0_2d, raw2d, emb_1d, lin_flat, bias16)
    return out.reshape(-1, 1)


# speedup vs baseline: 1.4894x; 1.0404x over previous
import jax
import jax.numpy as jnp
from jax import lax
from jax.experimental import pallas as pl
from jax.experimental.pallas import tpu as pltpu
from jax.experimental.pallas import tpu_sc as plsc

_F = 26
_V = 100000
_D = 32
_B = 16384

_L = 16
_NW = 32
_CB = _B // _NW
_GS = _L
_GROUPS = _CB // _GS
_CHUNK = 104
_CPG = _GS * _F // _CHUNK
_RG = _GS * _F
_CPW = _CB * _F // _CHUNK


def _make_fm_kernel():
    scmesh = plsc.VectorSubcoreMesh(core_axis_name="c", subcore_axis_name="s")

    def body(raw_hbm, emb_hbm, lin_hbm, bias_hbm, out_hbm,
             raw_v, dlist_v, llist_v, rows_v, lin_v, out_v, bias_v,
             sem_e0, sem_e1, sem_l0, sem_l1):
        c = lax.axis_index("c")
        s = lax.axis_index("s")
        wid = s * 2 + c
        pltpu.sync_copy(bias_hbm, bias_v)
        sems_e = (sem_e0, sem_e1)
        sems_l = (sem_l0, sem_l1)
        iota1 = lax.iota(jnp.int32, _L)

        def stage(g, par):
            pltpu.sync_copy(raw_hbm.at[pl.ds(wid * _GROUPS + g, 1), :],
                            raw_v.at[par])

        def build(par):
            dlist2 = dlist_v.at[par]
            llist2 = llist_v.at[par]

            def bb(k, carry):
                rv = raw_v[par, 0, pl.ds(k * _L, _L)]
                e = iota1 + k * _L
                rowi = e // _CHUNK
                coli = e % _CHUNK
                plsc.store_scatter(dlist2, [rowi, coli], rv >> 2)
                plsc.store_scatter(llist2, [rowi, coli], rv)
                return carry

            lax.fori_loop(0, _RG // _L, bb, 0)

        def fire(g, par):
            for j in range(_CPG):
                pltpu.async_copy(emb_hbm.at[dlist_v.at[par].at[j]],
                                 rows_v.at[par].at[pl.ds(j * _CHUNK, _CHUNK), :],
                                 sems_e[par])
                pltpu.async_copy(lin_hbm.at[llist_v.at[par].at[j]],
                                 lin_v.at[par].at[j],
                                 sems_l[par])

        def drain(par):
            for j in range(_CPG):
                pltpu.make_async_copy(emb_hbm.at[pl.ds(0, _CHUNK), :],
                                      rows_v.at[par].at[pl.ds(j * _CHUNK, _CHUNK), :],
                                      sems_e[par]).wait()
                pltpu.make_async_copy(lin_hbm.at[pl.ds(0, _CHUNK)],
                                      lin_v.at[par].at[j],
                                      sems_l[par]).wait()

        iota = lax.iota(jnp.int32, _L)
        iota26 = iota * _F
        zero = jnp.zeros((_L,), jnp.float32)

        def compute(g, par):
            rows2 = rows_v.at[par]
            raw2 = llist_v.at[par]
            lin2 = lin_v.at[par]

            def fbody(f, carry):
                accs = carry[:_D]
                acc_sq = carry[_D]
                lin_acc = carry[_D + 1]
                r = iota26 + f
                rc = r // _CHUNK
                rw = r % _CHUNK
                ivraw = plsc.load_gather(raw2, [rc, rw])
                colb = (ivraw & 3) * _D
                lin_acc = lin_acc + plsc.load_gather(lin2, [rc, rw])
                new_accs = []
                for d in range(_D):
                    v = plsc.load_gather(rows2, [r, colb + d])
                    new_accs.append(accs[d] + v)
                    acc_sq = acc_sq + v * v
                return (*new_accs, acc_sq, lin_acc)

            init = (zero,) * (_D + 2)
            res = lax.fori_loop(0, _F, fbody, init)
            accs, acc_sq, lin_acc = res[:_D], res[_D], res[_D + 1]
            ss = accs[0] * accs[0]
            for d in range(1, _D):
                ss = ss + accs[d] * accs[d]
            logit = bias_v[...] + lin_acc + 0.5 * (ss - acc_sq)
            out_v[pl.ds(g * _GS, _L)] = 1.0 / (1.0 + jnp.exp(-logit))

        stage(0, 0)
        build(0)
        fire(0, 0)
        stage(1, 1)
        build(1)

        def gbody(g2, carry):
            fire(2 * g2 + 1, 1)
            drain(0)
            compute(2 * g2, 0)

            @pl.when(g2 < _GROUPS // 2 - 1)
            def _():
                stage(2 * g2 + 2, 0)
                build(0)
                fire(2 * g2 + 2, 0)

            drain(1)
            compute(2 * g2 + 1, 1)

            @pl.when(g2 < _GROUPS // 2 - 1)
            def _():
                stage(2 * g2 + 3, 1)
                build(1)

            return carry

        lax.fori_loop(0, _GROUPS // 2, gbody, 0)
        pltpu.sync_copy(out_v, out_hbm.at[pl.ds(wid * _CB, _CB)])

    return pl.kernel(
        body,
        out_type=jax.ShapeDtypeStruct((_B,), jnp.float32),
        mesh=scmesh,
        scratch_types=[
            pltpu.VMEM((2, 1, _RG), jnp.int32),
            pltpu.VMEM((2, _CPG, _CHUNK), jnp.int32),
            pltpu.VMEM((2, _CPG, _CHUNK), jnp.int32),
            pltpu.VMEM((2, _RG, 4 * _D), jnp.float32),
            pltpu.VMEM((2, _CPG, _CHUNK), jnp.float32),
            pltpu.VMEM((_CB,), jnp.float32),
            pltpu.VMEM((_L,), jnp.float32),
            pltpu.SemaphoreType.DMA,
            pltpu.SemaphoreType.DMA,
            pltpu.SemaphoreType.DMA,
            pltpu.SemaphoreType.DMA,
        ],
        compiler_params=pltpu.CompilerParams(needs_layout_passes=False),
    )


def kernel(x, emb_tables, lin_tables, bias):
    F, V, D = emb_tables.shape
    idx = (x + jnp.arange(F, dtype=jnp.int32)[None, :] * V).reshape(-1)
    raw2d = idx.reshape(-1, _RG)
    emb_wide = emb_tables.reshape(F * V // 4, 4 * D)
    lin_flat = lin_tables.reshape(F * V)
    bias16 = jnp.broadcast_to(bias.astype(jnp.float32), (_L,))
    out = _make_fm_kernel()(raw2d, emb_wide, lin_flat, bias16)
    return out.reshape(-1, 1)


# speedup vs baseline: 1.4906x; 1.0008x over previous
import jax
import jax.numpy as jnp
from jax import lax
from jax.experimental import pallas as pl
from jax.experimental.pallas import tpu as pltpu
from jax.experimental.pallas import tpu_sc as plsc

_F = 26
_V = 100000
_D = 32
_B = 16384

_L = 16
_NW = 32
_CB = _B // _NW
_GS = _L
_GROUPS = _CB // _GS
_CHUNK = 104
_CPG = _GS * _F // _CHUNK
_RG = _GS * _F
_CPW = _CB * _F // _CHUNK


def _make_fm_kernel():
    scmesh = plsc.VectorSubcoreMesh(core_axis_name="c", subcore_axis_name="s")

    def body(dma_hbm, raw_hbm, emb_hbm, lin_hbm, bias_hbm, out_hbm,
             dma_v, raw_v, rows_v, lin_v, out_v, bias_v,
             sem_e0, sem_e1, sem_l0, sem_l1):
        c = lax.axis_index("c")
        s = lax.axis_index("s")
        wid = s * 2 + c
        pltpu.sync_copy(dma_hbm.at[pl.ds(wid * _CPW, _CPW), :], dma_v)
        pltpu.sync_copy(bias_hbm, bias_v)
        sems_e = (sem_e0, sem_e1)
        sems_l = (sem_l0, sem_l1)

        def fire(g, par):
            pltpu.sync_copy(raw_hbm.at[pl.ds(wid * _CPW + g * _CPG, _CPG), :],
                            raw_v.at[par])
            for j in range(_CPG):
                ch = g * _CPG + j
                pltpu.async_copy(emb_hbm.at[dma_v.at[ch]],
                                 rows_v.at[par].at[pl.ds(j * _CHUNK, _CHUNK), :],
                                 sems_e[par])
                pltpu.async_copy(lin_hbm.at[raw_v.at[par].at[j]],
                                 lin_v.at[par].at[j],
                                 sems_l[par])

        def drain(par):
            for j in range(_CPG):
                pltpu.make_async_copy(emb_hbm.at[pl.ds(0, _CHUNK), :],
                                      rows_v.at[par].at[pl.ds(j * _CHUNK, _CHUNK), :],
                                      sems_e[par]).wait()
                pltpu.make_async_copy(lin_hbm.at[pl.ds(0, _CHUNK)],
                                      lin_v.at[par].at[j],
                                      sems_l[par]).wait()

        iota = lax.iota(jnp.int32, _L)
        iota26 = iota * _F
        zero = jnp.zeros((_L,), jnp.float32)

        def compute(g, par):
            rows2 = rows_v.at[par]
            raw2 = raw_v.at[par]
            lin2 = lin_v.at[par]

            def fbody(f, carry):
                accs = carry[:_D]
                acc_sq = carry[_D]
                lin_acc = carry[_D + 1]
                r = iota26 + f
                rc = r // _CHUNK
                rw = r % _CHUNK
                ivraw = plsc.load_gather(raw2, [rc, rw])
                colb = (ivraw & 3) * _D
                lin_acc = lin_acc + plsc.load_gather(lin2, [rc, rw])
                new_accs = []
                for d in range(_D):
                    v = plsc.load_gather(rows2, [r, colb + d])
                    new_accs.append(accs[d] + v)
                    acc_sq = acc_sq + v * v
                return (*new_accs, acc_sq, lin_acc)

            init = (zero,) * (_D + 2)
            res = lax.fori_loop(0, _F, fbody, init)
            accs, acc_sq, lin_acc = res[:_D], res[_D], res[_D + 1]
            ss = accs[0] * accs[0]
            for d in range(1, _D):
                ss = ss + accs[d] * accs[d]
            logit = bias_v[...] + lin_acc + 0.5 * (ss - acc_sq)
            out_v[pl.ds(g * _GS, _L)] = 1.0 / (1.0 + jnp.exp(-logit))

        fire(0, 0)

        def gbody(g2, carry):
            fire(2 * g2 + 1, 1)
            drain(0)
            compute(2 * g2, 0)

            @pl.when(g2 < _GROUPS // 2 - 1)
            def _():
                fire(2 * g2 + 2, 0)

            drain(1)
            compute(2 * g2 + 1, 1)
            return carry

        lax.fori_loop(0, _GROUPS // 2, gbody, 0)
        pltpu.sync_copy(out_v, out_hbm.at[pl.ds(wid * _CB, _CB)])

    return pl.kernel(
        body,
        out_type=jax.ShapeDtypeStruct((_B,), jnp.float32),
        mesh=scmesh,
        scratch_types=[
            pltpu.VMEM((_CPW, _CHUNK), jnp.int32),
            pltpu.VMEM((2, _CPG, _CHUNK), jnp.int32),
            pltpu.VMEM((2, _RG, 4 * _D), jnp.float32),
            pltpu.VMEM((2, _CPG, _CHUNK), jnp.float32),
            pltpu.VMEM((_CB,), jnp.float32),
            pltpu.VMEM((_L,), jnp.float32),
            pltpu.SemaphoreType.DMA,
            pltpu.SemaphoreType.DMA,
            pltpu.SemaphoreType.DMA,
            pltpu.SemaphoreType.DMA,
        ],
        compiler_params=pltpu.CompilerParams(needs_layout_passes=False),
    )


def kernel(x, emb_tables, lin_tables, bias):
    F, V, D = emb_tables.shape
    idx = (x + jnp.arange(F, dtype=jnp.int32)[None, :] * V).reshape(-1)
    raw2d = idx.reshape(-1, _CHUNK)
    dma2d = (idx >> 2).reshape(-1, _CHUNK)
    emb_wide = emb_tables.reshape(F * V // 4, 4 * D)
    lin_flat = lin_tables.reshape(F * V)
    bias16 = jnp.broadcast_to(bias.astype(jnp.float32), (_L,))
    out = _make_fm_kernel()(dma2d, raw2d, emb_wide, lin_flat, bias16)
    return out.reshape(-1, 1)
